# Initial kernel scaffold; baseline (speedup 1.0000x reference)
import jax
import jax.numpy as jnp
from jax.experimental import pallas as pl


def kernel(input, input_mask, condensed_weight, bias):
    raise NotImplementedError("write your pallas kernel here")


# speedup vs baseline: 12.8659x; 12.8659x over previous
import functools

import jax
import jax.numpy as jnp
from jax import lax
from jax.experimental import pallas as pl
from jax.experimental.pallas import tpu as pltpu
from jax.experimental.pallas import tpu_sc as plsc

IN_F = 4096
OUT_F = 4096
K_PAD = 416
LANES = 16
CHUNKS = K_PAD // LANES

NUM_CORES = 2
NUM_SUBCORES = 16
NUM_WORKERS = NUM_CORES * NUM_SUBCORES
ROWS_PER_WORKER = OUT_F // NUM_WORKERS


def _build_dense_body(mask_hbm, w_hbm, dense_hbm, idx_v, wv_v, row_v):
    wid = lax.axis_index("s") * NUM_CORES + lax.axis_index("c")
    row0 = wid * ROWS_PER_WORKER
    zeros16 = jnp.zeros((LANES,), jnp.float32)

    def zinit(i, carry):
        row_v[pl.ds(i * LANES, LANES)] = zeros16
        return carry

    lax.fori_loop(0, IN_F // LANES, zinit, 0)

    def per_row(r, carry):
        o = row0 + r
        pltpu.sync_copy(mask_hbm.at[o], idx_v)
        pltpu.sync_copy(w_hbm.at[o], wv_v)

        def chunk_add(c, carry2):
            idx = idx_v[pl.ds(c * LANES, LANES)]
            wv = wv_v[pl.ds(c * LANES, LANES)]
            plsc.addupdate_scatter(row_v, [idx], wv)
            return carry2

        lax.fori_loop(0, CHUNKS, chunk_add, 0)
        pltpu.sync_copy(row_v, dense_hbm.at[o])

        def chunk_zero(c, carry2):
            idx = idx_v[pl.ds(c * LANES, LANES)]
            plsc.store_scatter(row_v, [idx], zeros16)
            return carry2

        lax.fori_loop(0, CHUNKS, chunk_zero, 0)
        return carry

    lax.fori_loop(0, ROWS_PER_WORKER, per_row, 0)


def _build_dense(mask_p, w_p):
    mesh = plsc.VectorSubcoreMesh(core_axis_name="c", subcore_axis_name="s")
    return pl.kernel(
        _build_dense_body,
        mesh=mesh,
        out_type=jax.ShapeDtypeStruct((OUT_F, IN_F), jnp.float32),
        scratch_types=[
            pltpu.VMEM((K_PAD,), jnp.int32),
            pltpu.VMEM((K_PAD,), jnp.float32),
            pltpu.VMEM((IN_F,), jnp.float32),
        ],
        compiler_params=pltpu.CompilerParams(needs_layout_passes=False),
    )(mask_p, w_p)


BN = 512


def _mm_body(x_ref, w_ref, b_ref, o_ref):
    w = w_ref[...].astype(jnp.bfloat16)
    acc = lax.dot_general(
        x_ref[...], w, (((1,), (1,)), ((), ())),
        preferred_element_type=jnp.float32,
    )
    o_ref[...] = acc + b_ref[...]


def _matmul(x_bf16, dense_w, bias2d):
    tokens = x_bf16.shape[0]
    return pl.pallas_call(
        _mm_body,
        grid=(OUT_F // BN,),
        in_specs=[
            pl.BlockSpec((tokens, IN_F), lambda n: (0, 0)),
            pl.BlockSpec((BN, IN_F), lambda n: (n, 0)),
            pl.BlockSpec((1, BN), lambda n: (0, n)),
        ],
        out_specs=pl.BlockSpec((tokens, BN), lambda n: (0, n)),
        out_shape=jax.ShapeDtypeStruct((tokens, OUT_F), jnp.float32),
        compiler_params=pltpu.CompilerParams(
            dimension_semantics=("arbitrary",),
        ),
    )(x_bf16, dense_w, bias2d)


def kernel(input, input_mask, condensed_weight, bias):
    k = input_mask.shape[1]
    pad = K_PAD - k
    mask_p = jnp.pad(input_mask, ((0, 0), (0, pad)))
    w_p = jnp.pad(condensed_weight, ((0, 0), (0, pad)))
    dense_w = _build_dense(mask_p, w_p)
    x_bf16 = input.astype(jnp.bfloat16)
    return _matmul(x_bf16, dense_w, bias.reshape(1, OUT_F))


# speedup vs baseline: 17.1821x; 1.3355x over previous
import jax
import jax.numpy as jnp
from jax import lax
from jax.experimental import pallas as pl
from jax.experimental.pallas import tpu as pltpu
from jax.experimental.pallas import tpu_sc as plsc

IN_F = 4096
OUT_F = 4096
K_NNZ = 412
LANES = 16
FULL_CHUNKS = K_NNZ // LANES
TAIL = K_NNZ - FULL_CHUNKS * LANES
CHUNKS = FULL_CHUNKS + 1
K_PAD = CHUNKS * LANES

NUM_CORES = 2
NUM_SUBCORES = 16
NUM_WORKERS = NUM_CORES * NUM_SUBCORES
ROWS_PER_WORKER = OUT_F // NUM_WORKERS
GROUP_ROWS = 8
GROUPS = ROWS_PER_WORKER // GROUP_ROWS
SLAB = GROUP_ROWS * K_NNZ
SLAB_PAD = SLAB + LANES
ACC_N = GROUP_ROWS * IN_F


def _build_dense_body(mask_hbm, w_hbm, dense_hbm,
                      idx0, idx1, wv0, wv1, tgt0, tgt1, acc0, acc1,
                      isem0, isem1, osem0, osem1):
    wid = lax.axis_index("s") * NUM_CORES + lax.axis_index("c")
    row0 = wid * ROWS_PER_WORKER
    zeros16 = jnp.zeros((LANES,), jnp.float32)
    tail_mask = lax.iota(jnp.int32, LANES) < TAIL
    bufs = ((idx0, wv0, tgt0, acc0, isem0, osem0),
            (idx1, wv1, tgt1, acc1, isem1, osem1))

    def in_copies(g, slot):
        idx_v, wv_v, _, _, isem, _ = bufs[slot]
        start = (row0 + g * GROUP_ROWS) * K_NNZ
        return (
            pltpu.make_async_copy(
                mask_hbm.at[pl.ds(start, SLAB)], idx_v.at[pl.ds(0, SLAB)],
                isem),
            pltpu.make_async_copy(
                w_hbm.at[pl.ds(start, SLAB)], wv_v.at[pl.ds(0, SLAB)],
                isem),
        )

    def out_copy(g, slot):
        acc_v, osem = bufs[slot][3], bufs[slot][5]
        start = (row0 + g * GROUP_ROWS) * IN_F
        return pltpu.make_async_copy(
            acc_v, dense_hbm.at[pl.ds(start, ACC_N)], osem)

    def zinit(i, carry):
        acc0[pl.ds(i * LANES, LANES)] = zeros16
        acc1[pl.ds(i * LANES, LANES)] = zeros16
        return carry

    lax.fori_loop(0, ACC_N // LANES, zinit, 0)

    for c in in_copies(0, 0) + in_copies(1, 1):
        c.start()

    def process(g, slot):
        idx_v, wv_v, tgt_v, acc_v, _, _ = bufs[slot]

        @pl.when(g >= 2)
        def _():
            out_copy(g - 2, slot).wait()
            for r in range(GROUP_ROWS):
                for c in range(FULL_CHUNKS):
                    tgt = tgt_v[pl.ds(r * K_PAD + c * LANES, LANES)]
                    plsc.store_scatter(acc_v, [tgt], zeros16)
                tgt = tgt_v[pl.ds(r * K_PAD + FULL_CHUNKS * LANES, LANES)]
                plsc.store_scatter(acc_v, [tgt], zeros16, mask=tail_mask)

        for c in in_copies(g, slot):
            c.wait()

        for r in range(GROUP_ROWS):
            rbase = r * IN_F
            for c in range(CHUNKS):
                off = r * K_NNZ + c * LANES
                idx = idx_v[pl.ds(off, LANES)]
                wv = wv_v[pl.ds(off, LANES)]
                tgt = rbase + idx
                tgt_v[pl.ds(r * K_PAD + c * LANES, LANES)] = tgt
                if c < FULL_CHUNKS:
                    plsc.addupdate_scatter(acc_v, [tgt], wv)
                else:
                    plsc.addupdate_scatter(acc_v, [tgt], wv, mask=tail_mask)

        @pl.when(g + 2 < GROUPS)
        def _():
            for c in in_copies(g + 2, slot):
                c.start()

        out_copy(g, slot).start()

    def per_pair(j, carry):
        process(2 * j, 0)
        process(2 * j + 1, 1)
        return carry

    lax.fori_loop(0, GROUPS // 2, per_pair, 0)

    out_copy(GROUPS - 2, 0).wait()
    out_copy(GROUPS - 1, 1).wait()


def _build_dense(mask_flat, w_flat):
    mesh = plsc.VectorSubcoreMesh(core_axis_name="c", subcore_axis_name="s")
    return pl.kernel(
        _build_dense_body,
        mesh=mesh,
        out_type=jax.ShapeDtypeStruct((OUT_F * IN_F,), jnp.float32),
        scratch_types=[
            pltpu.VMEM((SLAB_PAD,), jnp.int32),
            pltpu.VMEM((SLAB_PAD,), jnp.int32),
            pltpu.VMEM((SLAB_PAD,), jnp.float32),
            pltpu.VMEM((SLAB_PAD,), jnp.float32),
            pltpu.VMEM((GROUP_ROWS * K_PAD,), jnp.int32),
            pltpu.VMEM((GROUP_ROWS * K_PAD,), jnp.int32),
            pltpu.VMEM((ACC_N,), jnp.float32),
            pltpu.VMEM((ACC_N,), jnp.float32),
            pltpu.SemaphoreType.DMA,
            pltpu.SemaphoreType.DMA,
            pltpu.SemaphoreType.DMA,
            pltpu.SemaphoreType.DMA,
        ],
        compiler_params=pltpu.CompilerParams(needs_layout_passes=False),
    )(mask_flat, w_flat)


BN = 512


def _mm_body(x_ref, w_ref, b_ref, o_ref):
    w = w_ref[...].astype(jnp.bfloat16)
    acc = lax.dot_general(
        x_ref[...], w, (((1,), (1,)), ((), ())),
        preferred_element_type=jnp.float32,
    )
    o_ref[...] = acc + b_ref[...]


def _matmul(x_bf16, dense_w, bias2d):
    tokens = x_bf16.shape[0]
    return pl.pallas_call(
        _mm_body,
        grid=(OUT_F // BN,),
        in_specs=[
            pl.BlockSpec((tokens, IN_F), lambda n: (0, 0)),
            pl.BlockSpec((BN, IN_F), lambda n: (n, 0)),
            pl.BlockSpec((1, BN), lambda n: (0, n)),
        ],
        out_specs=pl.BlockSpec((tokens, BN), lambda n: (0, n)),
        out_shape=jax.ShapeDtypeStruct((tokens, OUT_F), jnp.float32),
        compiler_params=pltpu.CompilerParams(
            dimension_semantics=("arbitrary",),
        ),
    )(x_bf16, dense_w, bias2d)


def kernel(input, input_mask, condensed_weight, bias):
    mask_flat = input_mask.reshape(-1)
    w_flat = condensed_weight.reshape(-1)
    dense_w = _build_dense(mask_flat, w_flat).reshape(OUT_F, IN_F)
    x_bf16 = input.astype(jnp.bfloat16)
    return _matmul(x_bf16, dense_w, bias.reshape(1, OUT_F))
